# baseline (speedup 1.0000x reference)
import functools

import jax
import jax.numpy as jnp
from jax import lax
from jax.experimental import pallas as pl
from jax.experimental.pallas import tpu as pltpu
from jax.experimental.pallas import tpu_sc as plsc

_BM = 1024
_L = 16


def _logits_kernel(x_ref, w1_ref, b1_ref, w2_ref, b2_ref, lt_ref):
    h = jnp.dot(x_ref[...], w1_ref[...], preferred_element_type=jnp.float32)
    h = jnp.maximum(h + b1_ref[...], 0.0)
    logits = jnp.dot(h, w2_ref[...], preferred_element_type=jnp.float32)
    logits = logits + b2_ref[...]
    lt_ref[...] = logits.T


def _tc_logits_t(x, W1, b1, W2, b2):
    m, k = x.shape
    e = W2.shape[1]
    return pl.pallas_call(
        _logits_kernel,
        grid=(m // _BM,),
        in_specs=[
            pl.BlockSpec((_BM, k), lambda i: (i, 0)),
            pl.BlockSpec((k, k), lambda i: (0, 0)),
            pl.BlockSpec((1, k), lambda i: (0, 0)),
            pl.BlockSpec((k, e), lambda i: (0, 0)),
            pl.BlockSpec((1, e), lambda i: (0, 0)),
        ],
        out_specs=pl.BlockSpec((e, _BM), lambda i: (0, i)),
        out_shape=jax.ShapeDtypeStruct((e, m), jnp.float32),
    )(x, W1, b1.reshape(1, k), W2, b2.reshape(1, e))


def _route_body(n_exp, t_per_w, lt_hbm, rw_hbm, idx_hbm, lt_v, rw_v, idx_v):
    wid = lax.axis_index("s") * 2 + lax.axis_index("c")
    base = wid * t_per_w
    pltpu.sync_copy(lt_hbm.at[:, pl.ds(base, t_per_w)], lt_v)

    def group(g, carry):
        del carry
        off = g * _L
        iota = lax.iota(jnp.int32, _L)
        best = lt_v[0, pl.ds(off, _L)]
        bidx = jnp.zeros((_L,), jnp.int32)
        for ex in range(1, n_exp):
            v = lt_v[ex, pl.ds(off, _L)]
            c = v > best
            best = jnp.where(c, v, best)
            bidx = jnp.where(c, jnp.full((_L,), ex, jnp.int32), bidx)
        second = jnp.full((_L,), -jnp.inf, jnp.float32)
        sidx = jnp.zeros((_L,), jnp.int32)
        for ex in range(n_exp):
            v = lt_v[ex, pl.ds(off, _L)]
            evec = jnp.full((_L,), ex, jnp.int32)
            c = jnp.logical_and(v > second, bidx != evec)
            second = jnp.where(c, v, second)
            sidx = jnp.where(c, evec, sidx)
        rw1 = 1.0 / (1.0 + jnp.exp(second - best))
        rw_v[0, pl.ds(off, _L)] = rw1
        rw_v[1, pl.ds(off, _L)] = 1.0 - rw1
        idx_v[0, pl.ds(off, _L)] = bidx
        idx_v[1, pl.ds(off, _L)] = sidx
        return 0

    lax.fori_loop(0, t_per_w // _L, group, 0)
    pltpu.sync_copy(rw_v, rw_hbm.at[:, pl.ds(base, t_per_w)])
    pltpu.sync_copy(idx_v, idx_hbm.at[:, pl.ds(base, t_per_w)])


def _sc_route(logits_t):
    n_exp, m = logits_t.shape
    t_per_w = m // 32
    mesh = plsc.VectorSubcoreMesh(core_axis_name="c", subcore_axis_name="s")
    body = functools.partial(_route_body, n_exp, t_per_w)
    return pl.kernel(
        body,
        mesh=mesh,
        out_type=[
            jax.ShapeDtypeStruct((2, m), jnp.float32),
            jax.ShapeDtypeStruct((2, m), jnp.int32),
        ],
        scratch_types=[
            pltpu.VMEM((n_exp, t_per_w), jnp.float32),
            pltpu.VMEM((2, t_per_w), jnp.float32),
            pltpu.VMEM((2, t_per_w), jnp.int32),
        ],
    )(logits_t)


def kernel(x, W1, b1, W2, b2):
    logits_t = _tc_logits_t(x, W1, b1, W2, b2)
    rw_t, idx_t = _sc_route(logits_t)
    return rw_t.T, idx_t.T


# speedup vs baseline: 1.1119x; 1.1119x over previous
import jax
import jax.numpy as jnp
from jax.experimental import pallas as pl
from jax.experimental.pallas import tpu as pltpu

_BM = 1024


def _key_decode(k):
    b = jnp.where(k >= 0, k, (~k) | jnp.int32(-2147483648))
    return jax.lax.bitcast_convert_type(b & ~jnp.int32(63), jnp.float32)


def _gating_kernel(x_ref, w1_ref, b1_ref, w2_ref, b2_ref, rw_ref, idx_ref):
    h = jnp.dot(x_ref[...], w1_ref[...], preferred_element_type=jnp.float32)
    h = jnp.maximum(h + b1_ref[...], 0.0)
    logits = jnp.dot(h, w2_ref[...], preferred_element_type=jnp.float32)
    logits = logits + b2_ref[...]

    b = jax.lax.bitcast_convert_type(logits, jnp.int32)
    k = jnp.where(b < 0, ~(b & jnp.int32(0x7FFFFFFF)), b)
    iota = jax.lax.broadcasted_iota(jnp.int32, logits.shape, 1)
    kk = (k & ~jnp.int32(63)) | (63 - iota)

    k1 = jnp.max(kk, axis=-1, keepdims=True)
    masked = jnp.where(kk == k1, jnp.int32(-2147483648), kk)
    k2 = jnp.max(masked, axis=-1, keepdims=True)

    i1 = 63 - (k1 & 63)
    i2 = 63 - (k2 & 63)
    l1 = _key_decode(k1)
    l2 = _key_decode(k2)

    rw1 = 1.0 / (1.0 + jnp.exp(l2 - l1))
    rw_ref[:, 0:1] = rw1
    rw_ref[:, 1:2] = 1.0 - rw1
    idx_ref[:, 0:1] = i1
    idx_ref[:, 1:2] = i2


def kernel(x, W1, b1, W2, b2):
    m, k = x.shape
    e = W2.shape[1]
    grid = (m // _BM,)
    rw, idx = pl.pallas_call(
        _gating_kernel,
        grid=grid,
        in_specs=[
            pl.BlockSpec((_BM, k), lambda i: (i, 0)),
            pl.BlockSpec((k, k), lambda i: (0, 0)),
            pl.BlockSpec((1, k), lambda i: (0, 0)),
            pl.BlockSpec((k, e), lambda i: (0, 0)),
            pl.BlockSpec((1, e), lambda i: (0, 0)),
        ],
        out_specs=[
            pl.BlockSpec((_BM, 2), lambda i: (i, 0)),
            pl.BlockSpec((_BM, 2), lambda i: (i, 0)),
        ],
        out_shape=[
            jax.ShapeDtypeStruct((m, 2), jnp.float32),
            jax.ShapeDtypeStruct((m, 2), jnp.int32),
        ],
        compiler_params=pltpu.CompilerParams(
            dimension_semantics=("parallel",),
        ),
    )(x, W1, b1.reshape(1, k), W2, b2.reshape(1, e))
    return rw, idx
